# Initial kernel scaffold; baseline (speedup 1.0000x reference)
import functools

import jax
import jax.numpy as jnp
from jax import lax
from jax.experimental import pallas as pl
from jax.experimental.pallas import tpu as pltpu
from jax.experimental.pallas import tpu_sc as plsc

NC = 2
NS = 16
NW = NC * NS
D = 128
F32 = jnp.float32


def _zero_vmem_128(ref):
    def body(i, _):
        for j in range(8):
            ref[i, pl.ds(j * 16, 16)] = jnp.zeros((16,), F32)
        return 0
    lax.fori_loop(0, 128, body, 0, unroll=False)


def _zero_spmem_slice(zb, sp_ref, row0, nrows):
    full, rem = nrows // 128, nrows % 128
    for k in range(full):
        pltpu.sync_copy(zb, sp_ref.at[pl.ds(row0 + k * 128, 128)])
    if rem:
        pltpu.sync_copy(zb.at[pl.ds(0, rem)], sp_ref.at[pl.ds(row0 + full * 128, rem)])


def _scatter_chunks(wid, src_hbm, dst_hbm, x_hbm, agg_sp, idx_s, idx_d, rows,
                    sem, e_per_tile, chunk, n_chunks):
    def body(j, _):
        base = wid * e_per_tile + j * chunk
        pltpu.sync_copy(src_hbm.at[pl.ds(base, chunk)], idx_s)
        pltpu.sync_copy(dst_hbm.at[pl.ds(base, chunk)], idx_d)
        pltpu.async_copy(x_hbm.at[idx_s], rows, sem).wait()
        pltpu.sync_copy(rows, agg_sp.at[idx_d], add=True)
        return 0
    lax.fori_loop(0, n_chunks, body, 0, unroll=False)


def _make_sc_stage_a(E1p, E2p):
    e1_pt = E1p // NW
    e2_pt = E2p // NW
    mesh = plsc.VectorSubcoreMesh(core_axis_name="c", subcore_axis_name="s")

    @functools.partial(
        pl.kernel,
        out_type=(
            jax.ShapeDtypeStruct((NC, 10240, D), F32),
            jax.ShapeDtypeStruct((NC, 128, D), F32),
        ),
        mesh=mesh,
        scratch_types=[
            pltpu.VMEM((128,), jnp.int32),
            pltpu.VMEM((128,), jnp.int32),
            pltpu.VMEM((128, D), F32),
            pltpu.VMEM((64,), jnp.int32),
            pltpu.VMEM((64,), jnp.int32),
            pltpu.VMEM((64, D), F32),
            pltpu.VMEM((128, D), F32),
            pltpu.VMEM_SHARED((10240, D), F32),
            pltpu.VMEM_SHARED((128, D), F32),
            pltpu.SemaphoreType.DMA,
        ],
    )
    def stage_a(src1, dst1, src2, dst2, xgoal, out1, out2,
                idx_s, idx_d, rows, idx_s2, idx_d2, rows2, zb, agg1_sp, agg2_sp, sem):
        c = lax.axis_index("c")
        s = lax.axis_index("s")
        wid = c * NS + s
        _zero_vmem_128(zb)
        _zero_spmem_slice(zb, agg1_sp, s * 640, 640)
        _zero_spmem_slice(zb, agg2_sp, s * 8, 8)
        plsc.subcore_barrier()
        _scatter_chunks(wid, src1, dst1, xgoal, agg1_sp, idx_s, idx_d, rows,
                        sem, e1_pt, 128, e1_pt // 128)
        _scatter_chunks(wid, src2, dst2, xgoal, agg2_sp, idx_s2, idx_d2, rows2,
                        sem, e2_pt, 64, e2_pt // 64)
        plsc.subcore_barrier()
        pltpu.sync_copy(agg1_sp.at[pl.ds(s * 640, 640)], out1.at[c, pl.ds(s * 640, 640)])
        pltpu.sync_copy(agg2_sp.at[pl.ds(s * 8, 8)], out2.at[c, pl.ds(s * 8, 8)])

    return stage_a


def _make_sc_stage_b():
    mesh = plsc.VectorSubcoreMesh(core_axis_name="c", subcore_axis_name="s")
    e3_pt = 32768 // NW
    e4_pt = 2048 // NW

    @functools.partial(
        pl.kernel,
        out_type=(
            jax.ShapeDtypeStruct((NC, 2048, D), F32),
            jax.ShapeDtypeStruct((NC, 2048, D), F32),
        ),
        mesh=mesh,
        scratch_types=[
            pltpu.VMEM((128,), jnp.int32),
            pltpu.VMEM((128,), jnp.int32),
            pltpu.VMEM((128,), F32),
            pltpu.VMEM((128, D), F32),
            pltpu.VMEM((D,), F32),
            pltpu.VMEM((D,), F32),
            pltpu.VMEM((64,), jnp.int32),
            pltpu.VMEM((64,), jnp.int32),
            pltpu.VMEM((64, D), F32),
            pltpu.VMEM((128, D), F32),
            pltpu.VMEM_SHARED((2048, D), F32),
            pltpu.VMEM_SHARED((2048, D), F32),
            pltpu.SemaphoreType.DMA,
        ],
    )
    def stage_b(src3, dst3, attr3, we0, be0, x11, src4, dst4, x12, out3, out4,
                idx_s, idx_d, attr_v, rows, we_v, be_v, idx_s4, idx_d4, rows4,
                zb, agg3_sp, agg4_sp, sem):
        c = lax.axis_index("c")
        s = lax.axis_index("s")
        wid = c * NS + s
        _zero_vmem_128(zb)
        _zero_spmem_slice(zb, agg3_sp, s * 128, 128)
        _zero_spmem_slice(zb, agg4_sp, s * 128, 128)
        pltpu.sync_copy(we0, we_v)
        pltpu.sync_copy(be0, be_v)
        plsc.subcore_barrier()

        def chunk3(j, _):
            base = wid * e3_pt + j * 128
            pltpu.sync_copy(src3.at[pl.ds(base, 128)], idx_s)
            pltpu.sync_copy(dst3.at[pl.ds(base, 128)], idx_d)
            pltpu.sync_copy(attr3.at[pl.ds(base, 128)], attr_v)
            pltpu.async_copy(x11.at[idx_s], rows, sem).wait()

            def edge(i, _):
                a = attr_v[i]
                for q in range(8):
                    sl = pl.ds(q * 16, 16)
                    v = rows[i, sl] + a * we_v[sl] + be_v[sl]
                    rows[i, sl] = jnp.maximum(v, 0.0)
                return 0
            lax.fori_loop(0, 128, edge, 0, unroll=False)
            pltpu.sync_copy(rows, agg3_sp.at[idx_d], add=True)
            return 0
        lax.fori_loop(0, e3_pt // 128, chunk3, 0, unroll=False)

        base4 = wid * e4_pt
        pltpu.sync_copy(src4.at[pl.ds(base4, 64)], idx_s4)
        pltpu.sync_copy(dst4.at[pl.ds(base4, 64)], idx_d4)
        pltpu.async_copy(x12.at[idx_s4], rows4, sem).wait()
        pltpu.sync_copy(rows4, agg4_sp.at[idx_d4], add=True)

        plsc.subcore_barrier()
        pltpu.sync_copy(agg3_sp.at[pl.ds(s * 128, 128)], out3.at[c, pl.ds(s * 128, 128)])
        pltpu.sync_copy(agg4_sp.at[pl.ds(s * 128, 128)], out4.at[c, pl.ds(s * 128, 128)])

    return stage_b


def _make_sc_stage_c():
    mesh = plsc.VectorSubcoreMesh(core_axis_name="c", subcore_axis_name="s")
    e_pt = 2048 // NW

    @functools.partial(
        pl.kernel,
        out_type=jax.ShapeDtypeStruct((NC, 2048, D), F32),
        mesh=mesh,
        scratch_types=[
            pltpu.VMEM((64,), jnp.int32),
            pltpu.VMEM((64,), jnp.int32),
            pltpu.VMEM((64, D), F32),
            pltpu.VMEM((128, D), F32),
            pltpu.VMEM_SHARED((2048, D), F32),
            pltpu.SemaphoreType.DMA,
        ],
    )
    def stage_c(src5, dst5, x22, out5, idx_s, idx_d, rows, zb, agg_sp, sem):
        c = lax.axis_index("c")
        s = lax.axis_index("s")
        wid = c * NS + s
        _zero_vmem_128(zb)
        _zero_spmem_slice(zb, agg_sp, s * 128, 128)
        plsc.subcore_barrier()
        base = wid * e_pt
        pltpu.sync_copy(src5.at[pl.ds(base, 64)], idx_s)
        pltpu.sync_copy(dst5.at[pl.ds(base, 64)], idx_d)
        pltpu.async_copy(x22.at[idx_s], rows, sem).wait()
        pltpu.sync_copy(rows, agg_sp.at[idx_d], add=True)
        plsc.subcore_barrier()
        pltpu.sync_copy(agg_sp.at[pl.ds(s * 128, 128)], out5.at[c, pl.ds(s * 128, 128)])

    return stage_c


def _mlp_body(x_ref, p_ref, w1_ref, b1_ref, w2_ref, b2_ref, o_ref):
    h = x_ref[...] + p_ref[0] + p_ref[1]
    h = jnp.maximum(jnp.dot(h, w1_ref[...], preferred_element_type=F32) + b1_ref[...], 0.0)
    o_ref[...] = jnp.dot(h, w2_ref[...], preferred_element_type=F32) + b2_ref[...]


def _mlp(x, partials, w1, b1, w2, b2, block_rows):
    n = x.shape[0]
    nb = n // block_rows
    return pl.pallas_call(
        _mlp_body,
        grid=(nb,),
        in_specs=[
            pl.BlockSpec((block_rows, D), lambda i: (i, 0)),
            pl.BlockSpec((NC, block_rows, D), lambda i: (0, i, 0)),
            pl.BlockSpec((D, D), lambda i: (0, 0)),
            pl.BlockSpec((1, D), lambda i: (0, 0)),
            pl.BlockSpec((D, D), lambda i: (0, 0)),
            pl.BlockSpec((1, D), lambda i: (0, 0)),
        ],
        out_specs=pl.BlockSpec((block_rows, D), lambda i: (i, 0)),
        out_shape=jax.ShapeDtypeStruct((n, D), F32),
    )(x, partials, w1, b1, w2, b2)


@jax.jit
def kernel(x_goal, x_obs, x_task, x_virtual, x_actor, edge_attr_obs_task, params,
           ei_goal_obs, ei_goal_virtual, ei_obs_task, ei_virtual_task, ei_task_actor):
    p = params
    E1, E1p = ei_goal_obs.shape[1], 323584
    E2, E2p = ei_goal_virtual.shape[1], 10240

    pad1 = E1p - E1
    src1 = jnp.concatenate([ei_goal_obs[0], jnp.zeros((pad1,), jnp.int32)])
    dst1 = jnp.concatenate([ei_goal_obs[1], jnp.full((pad1,), 10000, jnp.int32)])
    pad2 = E2p - E2
    src2 = jnp.concatenate([ei_goal_virtual[0], jnp.zeros((pad2,), jnp.int32)])
    dst2 = jnp.concatenate([ei_goal_virtual[1], jnp.full((pad2,), 64, jnp.int32)])

    stage_a = _make_sc_stage_a(E1p, E2p)
    agg1_p, agg2_p = stage_a(src1, dst1, src2, dst2, x_goal)

    x_obs_pad = jnp.pad(x_obs, ((0, 240), (0, 0)))
    x11 = _mlp(x_obs_pad, agg1_p, p['ss']['W1'], p['ss']['b1'][None, :],
               p['ss']['W2'], p['ss']['b2'][None, :], 1280)
    x12 = _mlp(x_obs[:64], agg2_p[:, :64], p['sv']['W1'], p['sv']['b1'][None, :],
               p['sv']['W2'], p['sv']['b2'][None, :], 64)

    stage_b = _make_sc_stage_b()
    agg3_p, agg4_p = stage_b(ei_obs_task[0], ei_obs_task[1], edge_attr_obs_task[:, 0],
                             p['st']['We'][0], p['st']['be'], x11,
                             ei_virtual_task[0], ei_virtual_task[1], x12)

    x21 = _mlp(x_task, agg3_p, p['st']['W1'], p['st']['b1'][None, :],
               p['st']['W2'], p['st']['b2'][None, :], 1024)
    x22 = _mlp(x21, agg4_p, p['vt']['W1'], p['vt']['b1'][None, :],
               p['vt']['W2'], p['vt']['b2'][None, :], 1024)

    stage_c = _make_sc_stage_c()
    agg5_p = stage_c(ei_task_actor[0], ei_task_actor[1], x22)

    w2a = jnp.zeros((D, D), F32).at[:, :1].set(p['actor']['W2'])
    b2a = jnp.zeros((1, D), F32).at[0, 0].set(p['actor']['b2'][0])
    logits_full = _mlp(x_actor, agg5_p, p['actor']['W1'], p['actor']['b1'][None, :],
                       w2a, b2a, 1024)
    return logits_full[:, 0].reshape(64, 32)


# speedup vs baseline: 3.5619x; 3.5619x over previous
import functools

import jax
import jax.numpy as jnp
from jax import lax
from jax.experimental import pallas as pl
from jax.experimental.pallas import tpu as pltpu
from jax.experimental.pallas import tpu_sc as plsc

NC = 2
NS = 16
NW = NC * NS
D = 128
F32 = jnp.float32


def _zero_vmem_128(ref):
    def body(i, _):
        for j in range(8):
            ref[i, pl.ds(j * 16, 16)] = jnp.zeros((16,), F32)
        return 0
    lax.fori_loop(0, 128, body, 0, unroll=False)


def _zero_spmem_slice(zb, sp_ref, row0, nrows):
    full, rem = nrows // 128, nrows % 128
    for k in range(full):
        pltpu.sync_copy(zb, sp_ref.at[pl.ds(row0 + k * 128, 128)])
    if rem:
        pltpu.sync_copy(zb.at[pl.ds(0, rem)], sp_ref.at[pl.ds(row0 + full * 128, rem)])


def _scatter_chunks(wid, src_hbm, dst_hbm, x_hbm, agg_sp, idx_s, idx_d, rows,
                    sem, e_per_tile, chunk, n_chunks):
    def body(j, _):
        base = wid * e_per_tile + j * chunk
        pltpu.sync_copy(src_hbm.at[pl.ds(base, chunk)], idx_s)
        pltpu.sync_copy(dst_hbm.at[pl.ds(base, chunk)], idx_d)
        pltpu.async_copy(x_hbm.at[idx_s], rows, sem).wait()
        pltpu.sync_copy(rows, agg_sp.at[idx_d], add=True)
        return 0
    lax.fori_loop(0, n_chunks, body, 0, unroll=False)


def _make_sc_stage_a(E1p, E2p):
    e1_pt = E1p // NW
    e2_pt = E2p // NW
    mesh = plsc.VectorSubcoreMesh(core_axis_name="c", subcore_axis_name="s")

    @functools.partial(
        pl.kernel,
        out_type=(
            jax.ShapeDtypeStruct((NC, 10240, D), F32),
            jax.ShapeDtypeStruct((NC, 128, D), F32),
        ),
        mesh=mesh,
        scratch_types=[
            pltpu.VMEM((128,), jnp.int32),
            pltpu.VMEM((128,), jnp.int32),
            pltpu.VMEM((128, D), F32),
            pltpu.VMEM((64,), jnp.int32),
            pltpu.VMEM((64,), jnp.int32),
            pltpu.VMEM((64, D), F32),
            pltpu.VMEM((128, D), F32),
            pltpu.VMEM_SHARED((10240, D), F32),
            pltpu.VMEM_SHARED((128, D), F32),
            pltpu.SemaphoreType.DMA,
        ],
    )
    def stage_a(src1, dst1, src2, dst2, xgoal, out1, out2,
                idx_s, idx_d, rows, idx_s2, idx_d2, rows2, zb, agg1_sp, agg2_sp, sem):
        c = lax.axis_index("c")
        s = lax.axis_index("s")
        wid = c * NS + s
        _zero_vmem_128(zb)
        _zero_spmem_slice(zb, agg1_sp, s * 640, 640)
        _zero_spmem_slice(zb, agg2_sp, s * 8, 8)
        plsc.subcore_barrier()
        _scatter_chunks(wid, src1, dst1, xgoal, agg1_sp, idx_s, idx_d, rows,
                        sem, e1_pt, 128, e1_pt // 128)
        _scatter_chunks(wid, src2, dst2, xgoal, agg2_sp, idx_s2, idx_d2, rows2,
                        sem, e2_pt, 64, e2_pt // 64)
        plsc.subcore_barrier()
        pltpu.sync_copy(agg1_sp.at[pl.ds(s * 640, 640)], out1.at[c, pl.ds(s * 640, 640)])
        pltpu.sync_copy(agg2_sp.at[pl.ds(s * 8, 8)], out2.at[c, pl.ds(s * 8, 8)])

    return stage_a


def _make_sc_stage_b():
    mesh = plsc.VectorSubcoreMesh(core_axis_name="c", subcore_axis_name="s")
    e3_pt = 32768 // NW
    e4_pt = 2048 // NW

    @functools.partial(
        pl.kernel,
        out_type=(
            jax.ShapeDtypeStruct((NC, 2048, D), F32),
            jax.ShapeDtypeStruct((NC, 2048, D), F32),
        ),
        mesh=mesh,
        scratch_types=[
            pltpu.VMEM((128,), jnp.int32),
            pltpu.VMEM((128,), jnp.int32),
            pltpu.VMEM((128,), F32),
            pltpu.VMEM((128, D), F32),
            pltpu.VMEM((D,), F32),
            pltpu.VMEM((D,), F32),
            pltpu.VMEM((64,), jnp.int32),
            pltpu.VMEM((64,), jnp.int32),
            pltpu.VMEM((64, D), F32),
            pltpu.VMEM((128, D), F32),
            pltpu.VMEM_SHARED((2048, D), F32),
            pltpu.VMEM_SHARED((2048, D), F32),
            pltpu.SemaphoreType.DMA,
        ],
    )
    def stage_b(src3, dst3, attr3, we0, be0, x11, src4, dst4, x12, out3, out4,
                idx_s, idx_d, attr_v, rows, we_v, be_v, idx_s4, idx_d4, rows4,
                zb, agg3_sp, agg4_sp, sem):
        c = lax.axis_index("c")
        s = lax.axis_index("s")
        wid = c * NS + s
        _zero_vmem_128(zb)
        _zero_spmem_slice(zb, agg3_sp, s * 128, 128)
        _zero_spmem_slice(zb, agg4_sp, s * 128, 128)
        pltpu.sync_copy(we0, we_v)
        pltpu.sync_copy(be0, be_v)
        plsc.subcore_barrier()

        def chunk3(j, _):
            base = wid * e3_pt + j * 128
            pltpu.sync_copy(src3.at[pl.ds(base, 128)], idx_s)
            pltpu.sync_copy(dst3.at[pl.ds(base, 128)], idx_d)
            pltpu.sync_copy(attr3.at[pl.ds(base, 128)], attr_v)
            pltpu.async_copy(x11.at[idx_s], rows, sem).wait()

            def edge_group(g, _):
                va = attr_v[pl.ds(g * 16, 16)]
                for t in range(16):
                    i = g * 16 + t
                    a = va[t]
                    for q in range(8):
                        sl = pl.ds(q * 16, 16)
                        v = rows[i, sl] + a * we_v[sl] + be_v[sl]
                        rows[i, sl] = jnp.maximum(v, 0.0)
                return 0
            lax.fori_loop(0, 8, edge_group, 0, unroll=False)
            pltpu.sync_copy(rows, agg3_sp.at[idx_d], add=True)
            return 0
        lax.fori_loop(0, e3_pt // 128, chunk3, 0, unroll=False)

        base4 = wid * e4_pt
        pltpu.sync_copy(src4.at[pl.ds(base4, 64)], idx_s4)
        pltpu.sync_copy(dst4.at[pl.ds(base4, 64)], idx_d4)
        pltpu.async_copy(x12.at[idx_s4], rows4, sem).wait()
        pltpu.sync_copy(rows4, agg4_sp.at[idx_d4], add=True)

        plsc.subcore_barrier()
        pltpu.sync_copy(agg3_sp.at[pl.ds(s * 128, 128)], out3.at[c, pl.ds(s * 128, 128)])
        pltpu.sync_copy(agg4_sp.at[pl.ds(s * 128, 128)], out4.at[c, pl.ds(s * 128, 128)])

    return stage_b


def _make_sc_stage_c():
    mesh = plsc.VectorSubcoreMesh(core_axis_name="c", subcore_axis_name="s")
    e_pt = 2048 // NW

    @functools.partial(
        pl.kernel,
        out_type=jax.ShapeDtypeStruct((NC, 2048, D), F32),
        mesh=mesh,
        scratch_types=[
            pltpu.VMEM((64,), jnp.int32),
            pltpu.VMEM((64,), jnp.int32),
            pltpu.VMEM((64, D), F32),
            pltpu.VMEM((128, D), F32),
            pltpu.VMEM_SHARED((2048, D), F32),
            pltpu.SemaphoreType.DMA,
        ],
    )
    def stage_c(src5, dst5, x22, out5, idx_s, idx_d, rows, zb, agg_sp, sem):
        c = lax.axis_index("c")
        s = lax.axis_index("s")
        wid = c * NS + s
        _zero_vmem_128(zb)
        _zero_spmem_slice(zb, agg_sp, s * 128, 128)
        plsc.subcore_barrier()
        base = wid * e_pt
        pltpu.sync_copy(src5.at[pl.ds(base, 64)], idx_s)
        pltpu.sync_copy(dst5.at[pl.ds(base, 64)], idx_d)
        pltpu.async_copy(x22.at[idx_s], rows, sem).wait()
        pltpu.sync_copy(rows, agg_sp.at[idx_d], add=True)
        plsc.subcore_barrier()
        pltpu.sync_copy(agg_sp.at[pl.ds(s * 128, 128)], out5.at[c, pl.ds(s * 128, 128)])

    return stage_c


def _mlp_body(x_ref, p_ref, w1_ref, b1_ref, w2_ref, b2_ref, o_ref):
    h = x_ref[...] + p_ref[0] + p_ref[1]
    h = jnp.maximum(jnp.dot(h, w1_ref[...], preferred_element_type=F32) + b1_ref[...], 0.0)
    o_ref[...] = jnp.dot(h, w2_ref[...], preferred_element_type=F32) + b2_ref[...]


def _mlp(x, partials, w1, b1, w2, b2, block_rows):
    n = x.shape[0]
    nb = n // block_rows
    return pl.pallas_call(
        _mlp_body,
        grid=(nb,),
        in_specs=[
            pl.BlockSpec((block_rows, D), lambda i: (i, 0)),
            pl.BlockSpec((NC, block_rows, D), lambda i: (0, i, 0)),
            pl.BlockSpec((D, D), lambda i: (0, 0)),
            pl.BlockSpec((1, D), lambda i: (0, 0)),
            pl.BlockSpec((D, D), lambda i: (0, 0)),
            pl.BlockSpec((1, D), lambda i: (0, 0)),
        ],
        out_specs=pl.BlockSpec((block_rows, D), lambda i: (i, 0)),
        out_shape=jax.ShapeDtypeStruct((n, D), F32),
    )(x, partials, w1, b1, w2, b2)


@jax.jit
def kernel(x_goal, x_obs, x_task, x_virtual, x_actor, edge_attr_obs_task, params,
           ei_goal_obs, ei_goal_virtual, ei_obs_task, ei_virtual_task, ei_task_actor):
    p = params
    E1, E1p = ei_goal_obs.shape[1], 323584
    E2, E2p = ei_goal_virtual.shape[1], 10240

    pad1 = E1p - E1
    src1 = jnp.concatenate([ei_goal_obs[0], jnp.zeros((pad1,), jnp.int32)])
    dst1 = jnp.concatenate([ei_goal_obs[1], jnp.full((pad1,), 10000, jnp.int32)])
    pad2 = E2p - E2
    src2 = jnp.concatenate([ei_goal_virtual[0], jnp.zeros((pad2,), jnp.int32)])
    dst2 = jnp.concatenate([ei_goal_virtual[1], jnp.full((pad2,), 64, jnp.int32)])

    stage_a = _make_sc_stage_a(E1p, E2p)
    agg1_p, agg2_p = stage_a(src1, dst1, src2, dst2, x_goal)

    x_obs_pad = jnp.pad(x_obs, ((0, 240), (0, 0)))
    x11 = _mlp(x_obs_pad, agg1_p, p['ss']['W1'], p['ss']['b1'][None, :],
               p['ss']['W2'], p['ss']['b2'][None, :], 1280)
    x12 = _mlp(x_obs[:64], agg2_p[:, :64], p['sv']['W1'], p['sv']['b1'][None, :],
               p['sv']['W2'], p['sv']['b2'][None, :], 64)

    stage_b = _make_sc_stage_b()
    agg3_p, agg4_p = stage_b(ei_obs_task[0], ei_obs_task[1], edge_attr_obs_task[:, 0],
                             p['st']['We'][0], p['st']['be'], x11,
                             ei_virtual_task[0], ei_virtual_task[1], x12)

    x21 = _mlp(x_task, agg3_p, p['st']['W1'], p['st']['b1'][None, :],
               p['st']['W2'], p['st']['b2'][None, :], 1024)
    x22 = _mlp(x21, agg4_p, p['vt']['W1'], p['vt']['b1'][None, :],
               p['vt']['W2'], p['vt']['b2'][None, :], 1024)

    stage_c = _make_sc_stage_c()
    agg5_p = stage_c(ei_task_actor[0], ei_task_actor[1], x22)

    w2a = jnp.zeros((D, D), F32).at[:, :1].set(p['actor']['W2'])
    b2a = jnp.zeros((1, D), F32).at[0, 0].set(p['actor']['b2'][0])
    logits_full = _mlp(x_actor, agg5_p, p['actor']['W1'], p['actor']['b1'][None, :],
                       w2a, b2a, 1024)
    return logits_full[:, 0].reshape(64, 32)
